# Initial kernel scaffold; baseline (speedup 1.0000x reference)
import jax
import jax.numpy as jnp
from jax.experimental import pallas as pl


def kernel(edge_index, inputs, W_self0, W_neigh0, b0, gamma0, beta0, W_self1, W_neigh1, b1, gamma1, beta1, W_self2, W_neigh2, b2):
    raise NotImplementedError("write your pallas kernel here")


# speedup vs baseline: 6.3352x; 6.3352x over previous
import functools

import jax
import jax.numpy as jnp
from jax import lax
from jax.experimental import pallas as pl
from jax.experimental.pallas import tpu as pltpu
from jax.experimental.pallas import tpu_sc as plsc

NC = 2
NS = 16
NW = NC * NS
B = 128
L = 16


def _sc_mesh():
    return plsc.VectorSubcoreMesh(
        core_axis_name="c", subcore_axis_name="s", num_cores=NC,
        num_subcores=NS)


def _make_pass_a(n_pad, ch, d):
    rps = n_pad // NS

    @functools.partial(
        pl.kernel,
        out_type=[
            jax.ShapeDtypeStruct((NC, n_pad, d), jnp.float32),
            jax.ShapeDtypeStruct((NW, n_pad), jnp.float32),
        ],
        mesh=_sc_mesh(),
        compiler_params=pltpu.CompilerParams(needs_layout_passes=False),
        scratch_types=[
            pltpu.VMEM((B,), jnp.int32),
            pltpu.VMEM((B,), jnp.int32),
            pltpu.VMEM((B, d), jnp.float32),
            pltpu.VMEM((n_pad,), jnp.float32),
            pltpu.VMEM_SHARED((n_pad, d), jnp.float32),
            pltpu.SemaphoreType.DMA,
        ],
    )
    def pass_a(x_hbm, src_hbm, dst_hbm, zero_d_hbm, zero_1_hbm,
               sums_out, degv_out,
               idx_s, idx_d, rows, deg_v, sum_acc, sem):
        c = lax.axis_index("c")
        s = lax.axis_index("s")
        wid = s * NC + c
        stripe = pl.ds(s * rps, rps)

        pltpu.sync_copy(zero_d_hbm.at[stripe], sum_acc.at[stripe])
        pltpu.sync_copy(zero_1_hbm, deg_v)
        plsc.subcore_barrier()

        ones = jnp.ones((L,), jnp.float32)

        def step(k, _):
            pltpu.sync_copy(src_hbm.at[wid, k], idx_s)
            pltpu.sync_copy(dst_hbm.at[wid, k], idx_d)
            pltpu.async_copy(x_hbm.at[idx_s], rows, sem).wait()
            pltpu.sync_copy(rows, sum_acc.at[idx_d], add=True)

            def grp(g, _):
                dv = idx_d[pl.ds(g * L, L)]
                plsc.addupdate_scatter(deg_v, [dv], ones)
                return 0
            lax.fori_loop(0, B // L, grp, 0)
            return 0
        lax.fori_loop(0, ch, step, 0)

        plsc.subcore_barrier()
        pltpu.sync_copy(sum_acc.at[stripe], sums_out.at[c, stripe])
        pltpu.sync_copy(deg_v, degv_out.at[wid])

    return pass_a


def _make_pass_b(n_pad, ch, d):
    rps = n_pad // NS

    @functools.partial(
        pl.kernel,
        out_type=[
            jax.ShapeDtypeStruct((NC, n_pad, d), jnp.float32),
            jax.ShapeDtypeStruct((NW, n_pad), jnp.float32),
        ],
        mesh=_sc_mesh(),
        compiler_params=pltpu.CompilerParams(needs_layout_passes=False),
        scratch_types=[
            pltpu.VMEM((B,), jnp.int32),
            pltpu.VMEM((B,), jnp.int32),
            pltpu.VMEM((B, d), jnp.float32),
            pltpu.VMEM((n_pad,), jnp.float32),
            pltpu.VMEM((n_pad,), jnp.float32),
            pltpu.VMEM_SHARED((n_pad, d), jnp.float32),
            pltpu.SemaphoreType.DMA,
        ],
    )
    def pass_b(h_hbm, inv_hbm, src_hbm, dst_hbm, zero_d_hbm, zero_1_hbm,
               sums_out, cv_out,
               idx_s, idx_d, rows, inv_v, c_v, sum_acc, sem):
        c = lax.axis_index("c")
        s = lax.axis_index("s")
        wid = s * NC + c
        stripe = pl.ds(s * rps, rps)

        pltpu.sync_copy(zero_d_hbm.at[stripe], sum_acc.at[stripe])
        pltpu.sync_copy(zero_1_hbm, c_v)
        pltpu.sync_copy(inv_hbm, inv_v)
        plsc.subcore_barrier()

        def step(k, _):
            pltpu.sync_copy(src_hbm.at[wid, k], idx_s)
            pltpu.sync_copy(dst_hbm.at[wid, k], idx_d)
            pltpu.async_copy(h_hbm.at[idx_s], rows, sem).wait()
            pltpu.sync_copy(rows, sum_acc.at[idx_d], add=True)

            def grp(g, _):
                sv = idx_s[pl.ds(g * L, L)]
                dv = idx_d[pl.ds(g * L, L)]
                vals = plsc.load_gather(inv_v, [dv])
                plsc.addupdate_scatter(c_v, [sv], vals)
                return 0
            lax.fori_loop(0, B // L, grp, 0)
            return 0
        lax.fori_loop(0, ch, step, 0)

        plsc.subcore_barrier()
        pltpu.sync_copy(sum_acc.at[stripe], sums_out.at[c, stripe])
        pltpu.sync_copy(c_v, cv_out.at[wid])

    return pass_b


def _layer_body(make_inv, n, n_pad,
                x_ref, sums_ref, degs_ref, ws_ref, wn_ref, b_ref, g_ref,
                be_ref, *out_refs):
    x = x_ref[...]
    summed = sums_ref[0, :n, :] + sums_ref[1, :n, :]
    deg_full = jnp.sum(degs_ref[...], axis=0)
    deg = deg_full[:n, None]
    agg = jnp.where(deg > 0, summed / jnp.maximum(deg, 1.0), 0.0)
    t = (jnp.dot(x, ws_ref[...], preferred_element_type=jnp.float32)
         + jnp.dot(agg, wn_ref[...], preferred_element_type=jnp.float32)
         + b_ref[...])
    m = jnp.mean(t, axis=0, keepdims=True)
    v = jnp.mean(jnp.square(t - m), axis=0, keepdims=True)
    h = g_ref[...] * (t - m) * lax.rsqrt(v + 1e-5) + be_ref[...]
    out_refs[0][...] = jnp.maximum(h, 0.0)
    if make_inv:
        row = lax.iota(jnp.int32, n_pad)
        inv = jnp.where(row < n, 1.0 / jnp.maximum(deg_full, 1.0), 0.0)
        out_refs[1][...] = inv


def _final_body(n, feat_ref, cv_ref, ws_ref, wn_ref, b_ref, out_ref):
    feat = feat_ref[...]
    cvec = jnp.sum(cv_ref[...], axis=0)[:n, None]
    sacc = jnp.sum(feat * cvec, axis=0, keepdims=True)
    mf = jnp.mean(feat, axis=0, keepdims=True)
    o = (jnp.dot(mf, ws_ref[...], preferred_element_type=jnp.float32)
         + jnp.dot(sacc / n, wn_ref[...], preferred_element_type=jnp.float32)
         + b_ref[...])
    z = o - jnp.max(o, axis=-1, keepdims=True)
    out_ref[...] = z - jnp.log(jnp.sum(jnp.exp(z), axis=-1, keepdims=True))


def kernel(edge_index, inputs, W_self0, W_neigh0, b0, gamma0, beta0,
           W_self1, W_neigh1, b1, gamma1, beta1, W_self2, W_neigh2, b2):
    n, d = inputs.shape
    e = edge_index.shape[1]
    d_out = W_self2.shape[1]

    ch = -(-e // (NW * B))
    e_pad = ch * NW * B
    n_pad = -(-(n + 1) // (NS * 8)) * (NS * 8)
    src = edge_index[0]
    dst = edge_index[1]
    pad = e_pad - e
    if pad:
        src = jnp.concatenate([src, jnp.zeros((pad,), jnp.int32)])
        dst = jnp.concatenate([dst, jnp.full((pad,), n, jnp.int32)])
    src3 = src.reshape(NW, ch, B)
    dst3 = dst.reshape(NW, ch, B)
    zero_d = jnp.zeros((n_pad, d), jnp.float32)
    zero_1 = jnp.zeros((n_pad,), jnp.float32)

    pass_a = _make_pass_a(n_pad, ch, d)
    pass_b = _make_pass_b(n_pad, ch, d)

    def layer(x, sums, degs, ws, wn, b, g, be, make_inv):
        outs = [jax.ShapeDtypeStruct((n, d), jnp.float32)]
        if make_inv:
            outs.append(jax.ShapeDtypeStruct((n_pad,), jnp.float32))
        return pl.pallas_call(
            functools.partial(_layer_body, make_inv, n, n_pad),
            out_shape=outs,
        )(x, sums, degs, ws, wn, b, g, be)

    sums_a, degv = pass_a(inputs, src3, dst3, zero_d, zero_1)
    h1, invd = layer(inputs, sums_a, degv, W_self0, W_neigh0, b0, gamma0,
                     beta0, True)
    sums_b, cv = pass_b(h1, invd, src3, dst3, zero_d, zero_1)
    (feat,) = layer(h1, sums_b, degv, W_self1, W_neigh1, b1, gamma1,
                    beta1, False)
    out = pl.pallas_call(
        functools.partial(_final_body, n),
        out_shape=jax.ShapeDtypeStruct((1, d_out), jnp.float32),
    )(feat, cv, W_self2, W_neigh2, b2)
    return out, inputs, feat


# speedup vs baseline: 7.4387x; 1.1742x over previous
import functools

import jax
import jax.numpy as jnp
from jax import lax
from jax.experimental import pallas as pl
from jax.experimental.pallas import tpu as pltpu
from jax.experimental.pallas import tpu_sc as plsc

NC = 2
NS = 16
NW = NC * NS
B = 64
L = 16


def _sc_mesh():
    return plsc.VectorSubcoreMesh(
        core_axis_name="c", subcore_axis_name="s", num_cores=NC,
        num_subcores=NS)


def _make_pass_a(n_pad, ch, d):
    rps = n_pad // NS

    @functools.partial(
        pl.kernel,
        out_type=[
            jax.ShapeDtypeStruct((NC, n_pad, d), jnp.float32),
            jax.ShapeDtypeStruct((NW, n_pad), jnp.float32),
        ],
        mesh=_sc_mesh(),
        compiler_params=pltpu.CompilerParams(needs_layout_passes=False),
        scratch_types=[
            pltpu.VMEM((B,), jnp.int32),
            pltpu.VMEM((B,), jnp.int32),
            pltpu.VMEM((B,), jnp.int32),
            pltpu.VMEM((B,), jnp.int32),
            pltpu.VMEM((B, d), jnp.float32),
            pltpu.VMEM((B, d), jnp.float32),
            pltpu.VMEM((n_pad,), jnp.float32),
            pltpu.VMEM_SHARED((n_pad, d), jnp.float32),
            pltpu.SemaphoreType.DMA,
            pltpu.SemaphoreType.DMA,
        ],
    )
    def pass_a(x_hbm, src_hbm, dst_hbm, zero_d_hbm, zero_1_hbm,
               sums_out, degv_out,
               idx_s0, idx_d0, idx_s1, idx_d1, rows0, rows1, deg_v,
               sum_acc, sem0, sem1):
        c = lax.axis_index("c")
        s = lax.axis_index("s")
        wid = s * NC + c
        stripe = pl.ds(s * rps, rps)

        pltpu.sync_copy(zero_d_hbm.at[stripe], sum_acc.at[stripe])
        pltpu.sync_copy(zero_1_hbm, deg_v)
        plsc.subcore_barrier()

        ones = jnp.ones((L,), jnp.float32)
        dummy = zero_d_hbm.at[pl.ds(0, B)]

        def degc(idx_d):
            def grp(g, _):
                dv = idx_d[pl.ds(g * L, L)]
                plsc.addupdate_scatter(deg_v, [dv], ones)
                return 0
            lax.fori_loop(0, B // L, grp, 0)

        pltpu.sync_copy(src_hbm.at[wid, 0], idx_s0)
        pltpu.sync_copy(dst_hbm.at[wid, 0], idx_d0)
        pltpu.async_copy(x_hbm.at[idx_s0], rows0, sem0)

        def pair(p, _):
            a = 2 * p
            b = a + 1
            pltpu.sync_copy(src_hbm.at[wid, b], idx_s1)
            pltpu.sync_copy(dst_hbm.at[wid, b], idx_d1)
            pltpu.async_copy(x_hbm.at[idx_s1], rows1, sem1)
            degc(idx_d0)
            pltpu.make_async_copy(dummy, rows0, sem0).wait()
            pltpu.sync_copy(rows0, sum_acc.at[idx_d0], add=True)
            nxt = lax.rem(a + 2, ch)
            pltpu.sync_copy(src_hbm.at[wid, nxt], idx_s0)
            pltpu.sync_copy(dst_hbm.at[wid, nxt], idx_d0)
            pltpu.async_copy(x_hbm.at[idx_s0], rows0, sem0)
            degc(idx_d1)
            pltpu.make_async_copy(dummy, rows1, sem1).wait()
            pltpu.sync_copy(rows1, sum_acc.at[idx_d1], add=True)
            return 0
        lax.fori_loop(0, ch // 2, pair, 0)
        pltpu.make_async_copy(dummy, rows0, sem0).wait()

        plsc.subcore_barrier()
        pltpu.sync_copy(sum_acc.at[stripe], sums_out.at[c, stripe])
        pltpu.sync_copy(deg_v, degv_out.at[wid])

    return pass_a


def _make_pass_b(n_pad, ch, d):
    rps = n_pad // NS

    @functools.partial(
        pl.kernel,
        out_type=[
            jax.ShapeDtypeStruct((NC, n_pad, d), jnp.float32),
            jax.ShapeDtypeStruct((NW, n_pad), jnp.float32),
        ],
        mesh=_sc_mesh(),
        compiler_params=pltpu.CompilerParams(needs_layout_passes=False),
        scratch_types=[
            pltpu.VMEM((B,), jnp.int32),
            pltpu.VMEM((B,), jnp.int32),
            pltpu.VMEM((B,), jnp.int32),
            pltpu.VMEM((B,), jnp.int32),
            pltpu.VMEM((B, d), jnp.float32),
            pltpu.VMEM((B, d), jnp.float32),
            pltpu.VMEM((n_pad,), jnp.float32),
            pltpu.VMEM((n_pad,), jnp.float32),
            pltpu.VMEM_SHARED((n_pad, d), jnp.float32),
            pltpu.SemaphoreType.DMA,
            pltpu.SemaphoreType.DMA,
        ],
    )
    def pass_b(h_hbm, inv_hbm, src_hbm, dst_hbm, zero_d_hbm, zero_1_hbm,
               sums_out, cv_out,
               idx_s0, idx_d0, idx_s1, idx_d1, rows0, rows1, inv_v, c_v,
               sum_acc, sem0, sem1):
        c = lax.axis_index("c")
        s = lax.axis_index("s")
        wid = s * NC + c
        stripe = pl.ds(s * rps, rps)

        pltpu.sync_copy(zero_d_hbm.at[stripe], sum_acc.at[stripe])
        pltpu.sync_copy(zero_1_hbm, c_v)
        pltpu.sync_copy(inv_hbm, inv_v)
        plsc.subcore_barrier()

        dummy = zero_d_hbm.at[pl.ds(0, B)]

        def cupd(idx_s, idx_d):
            def grp(g, _):
                sv = idx_s[pl.ds(g * L, L)]
                dv = idx_d[pl.ds(g * L, L)]
                vals = plsc.load_gather(inv_v, [dv])
                plsc.addupdate_scatter(c_v, [sv], vals)
                return 0
            lax.fori_loop(0, B // L, grp, 0)

        pltpu.sync_copy(src_hbm.at[wid, 0], idx_s0)
        pltpu.sync_copy(dst_hbm.at[wid, 0], idx_d0)
        pltpu.async_copy(h_hbm.at[idx_s0], rows0, sem0)

        def pair(p, _):
            a = 2 * p
            b = a + 1
            pltpu.sync_copy(src_hbm.at[wid, b], idx_s1)
            pltpu.sync_copy(dst_hbm.at[wid, b], idx_d1)
            pltpu.async_copy(h_hbm.at[idx_s1], rows1, sem1)
            cupd(idx_s0, idx_d0)
            pltpu.make_async_copy(dummy, rows0, sem0).wait()
            pltpu.sync_copy(rows0, sum_acc.at[idx_d0], add=True)
            nxt = lax.rem(a + 2, ch)
            pltpu.sync_copy(src_hbm.at[wid, nxt], idx_s0)
            pltpu.sync_copy(dst_hbm.at[wid, nxt], idx_d0)
            pltpu.async_copy(h_hbm.at[idx_s0], rows0, sem0)
            cupd(idx_s1, idx_d1)
            pltpu.make_async_copy(dummy, rows1, sem1).wait()
            pltpu.sync_copy(rows1, sum_acc.at[idx_d1], add=True)
            return 0
        lax.fori_loop(0, ch // 2, pair, 0)
        pltpu.make_async_copy(dummy, rows0, sem0).wait()

        plsc.subcore_barrier()
        pltpu.sync_copy(sum_acc.at[stripe], sums_out.at[c, stripe])
        pltpu.sync_copy(c_v, cv_out.at[wid])

    return pass_b


def _layer_body(make_inv, n, n_pad,
                x_ref, sums_ref, degs_ref, ws_ref, wn_ref, b_ref, g_ref,
                be_ref, *out_refs):
    x = x_ref[...]
    summed = sums_ref[0, :n, :] + sums_ref[1, :n, :]
    deg_full = jnp.sum(degs_ref[...], axis=0)
    deg = deg_full[:n, None]
    agg = jnp.where(deg > 0, summed / jnp.maximum(deg, 1.0), 0.0)
    t = (jnp.dot(x, ws_ref[...], preferred_element_type=jnp.float32)
         + jnp.dot(agg, wn_ref[...], preferred_element_type=jnp.float32)
         + b_ref[...])
    m = jnp.mean(t, axis=0, keepdims=True)
    v = jnp.mean(jnp.square(t - m), axis=0, keepdims=True)
    h = g_ref[...] * (t - m) * lax.rsqrt(v + 1e-5) + be_ref[...]
    out_refs[0][...] = jnp.maximum(h, 0.0)
    if make_inv:
        row = lax.iota(jnp.int32, n_pad)
        inv = jnp.where(row < n, 1.0 / jnp.maximum(deg_full, 1.0), 0.0)
        out_refs[1][...] = inv


def _final_body(n, feat_ref, cv_ref, ws_ref, wn_ref, b_ref, out_ref):
    feat = feat_ref[...]
    cvec = jnp.sum(cv_ref[...], axis=0)[:n, None]
    sacc = jnp.sum(feat * cvec, axis=0, keepdims=True)
    mf = jnp.mean(feat, axis=0, keepdims=True)
    o = (jnp.dot(mf, ws_ref[...], preferred_element_type=jnp.float32)
         + jnp.dot(sacc / n, wn_ref[...], preferred_element_type=jnp.float32)
         + b_ref[...])
    z = o - jnp.max(o, axis=-1, keepdims=True)
    out_ref[...] = z - jnp.log(jnp.sum(jnp.exp(z), axis=-1, keepdims=True))


def kernel(edge_index, inputs, W_self0, W_neigh0, b0, gamma0, beta0,
           W_self1, W_neigh1, b1, gamma1, beta1, W_self2, W_neigh2, b2):
    n, d = inputs.shape
    e = edge_index.shape[1]
    d_out = W_self2.shape[1]

    ch = -(-e // (NW * B))
    ch += ch % 2
    e_pad = ch * NW * B
    n_pad = -(-(n + 1) // (NS * 8)) * (NS * 8)
    src = edge_index[0]
    dst = edge_index[1]
    pad = e_pad - e
    if pad:
        src = jnp.concatenate([src, jnp.zeros((pad,), jnp.int32)])
        dst = jnp.concatenate([dst, jnp.full((pad,), n, jnp.int32)])
    src3 = src.reshape(NW, ch, B)
    dst3 = dst.reshape(NW, ch, B)
    zero_d = jnp.zeros((n_pad, d), jnp.float32)
    zero_1 = jnp.zeros((n_pad,), jnp.float32)

    pass_a = _make_pass_a(n_pad, ch, d)
    pass_b = _make_pass_b(n_pad, ch, d)

    def layer(x, sums, degs, ws, wn, b, g, be, make_inv):
        outs = [jax.ShapeDtypeStruct((n, d), jnp.float32)]
        if make_inv:
            outs.append(jax.ShapeDtypeStruct((n_pad,), jnp.float32))
        return pl.pallas_call(
            functools.partial(_layer_body, make_inv, n, n_pad),
            out_shape=outs,
        )(x, sums, degs, ws, wn, b, g, be)

    sums_a, degv = pass_a(inputs, src3, dst3, zero_d, zero_1)
    h1, invd = layer(inputs, sums_a, degv, W_self0, W_neigh0, b0, gamma0,
                     beta0, True)
    sums_b, cv = pass_b(h1, invd, src3, dst3, zero_d, zero_1)
    (feat,) = layer(h1, sums_b, degv, W_self1, W_neigh1, b1, gamma1,
                    beta1, False)
    out = pl.pallas_call(
        functools.partial(_final_body, n),
        out_shape=jax.ShapeDtypeStruct((1, d_out), jnp.float32),
    )(feat, cv, W_self2, W_neigh2, b2)
    return out, inputs, feat
